# baseline (speedup 1.0000x reference)
import jax
import jax.numpy as jnp
from jax import lax
from jax.experimental import pallas as pl
from jax.experimental.pallas import tpu as pltpu
from jax.experimental.pallas import tpu_sc as plsc

_B, _C, _N = 2, 3, 4096
_S = 512
_TI = 512
_NT = (_N - _S) // _TI
_NC, _NS, _L = 2, 16, 16
_NW = _NC * _NS
_SPW = _S // _NW
_NJ = _N // _L
_HB = 8


def _sc_body(cl_hbm, nr_hbm, d1_hbm, d2_hbm,
             y_v, yn_v, x_v, xn_v, o_v, m_v):
    wid = lax.axis_index("s") * _NC + lax.axis_index("c")
    x0 = wid * _SPW
    lane = lax.iota(jnp.int32, _L)

    for d in range(2):
        for b in range(_B):
            pltpu.sync_copy(cl_hbm.at[1 - d, b], y_v)
            pltpu.sync_copy(nr_hbm.at[1 - d, b], yn_v)
            for c in range(_C):
                pltpu.sync_copy(cl_hbm.at[d, b, c, pl.ds(x0, _SPW)],
                                x_v.at[c])
            pltpu.sync_copy(nr_hbm.at[d, b, pl.ds(x0, _SPW)], xn_v)

            vx0 = x_v[0, :]
            vx1 = x_v[1, :]
            vx2 = x_v[2, :]
            for half in range(_SPW // _HB):
                xm = [tuple(jnp.broadcast_to(vx[half * _HB + ii] * (-2.0),
                                             (_L,))
                            for vx in (vx0, vx1, vx2))
                      for ii in range(_HB)]

                def jbody(off, mins):
                    y0 = y_v[0, pl.ds(off, _L)]
                    y1 = y_v[1, pl.ds(off, _L)]
                    y2 = y_v[2, pl.ds(off, _L)]
                    yn = yn_v[pl.ds(off, _L)]
                    return tuple(
                        jnp.minimum(mins[ii],
                                    yn + xm[ii][0] * y0
                                    + xm[ii][1] * y1
                                    + xm[ii][2] * y2)
                        for ii in range(_HB))

                init = tuple(jnp.full((_L,), 3.0e38, jnp.float32)
                             for _ in range(_HB))
                mins = plsc.parallel_loop(
                    0, _N, step=_L, unroll=2, carry=init)(jbody)
                for ii in range(_HB):
                    m_v[pl.ds((half * _HB + ii) * _L, _L)] = mins[ii]
            red = None
            for l in range(_L):
                col = plsc.load_gather(m_v, [lane * _L + l])
                red = col if red is None else jnp.minimum(red, col)
            o_v[...] = red + xn_v[...]
            out = d1_hbm if d == 0 else d2_hbm
            pltpu.sync_copy(o_v, out.at[b, pl.ds(x0, _SPW)])


_chamfer_sc = pl.kernel(
    _sc_body,
    out_type=(jax.ShapeDtypeStruct((_B, _S), jnp.float32),
              jax.ShapeDtypeStruct((_B, _S), jnp.float32)),
    mesh=plsc.VectorSubcoreMesh(core_axis_name="c", subcore_axis_name="s"),
    compiler_params=pltpu.CompilerParams(needs_layout_passes=False,
                                         skip_device_barrier=True),
    scratch_types=[
        pltpu.VMEM((_C, _N), jnp.float32),
        pltpu.VMEM((_N,), jnp.float32),
        pltpu.VMEM((_C, _SPW), jnp.float32),
        pltpu.VMEM((_SPW,), jnp.float32),
        pltpu.VMEM((_SPW,), jnp.float32),
        pltpu.VMEM((_L * _L,), jnp.float32),
    ],
)


def _tc_body(co_ref, cot_ref, sxt_ref, sy_ref, out_ref):
    yc = co_ref[0] + sy_ref[0, 0]
    yb = yc.astype(jnp.bfloat16)
    n2 = jnp.sum(yc * yc, axis=0)
    for k in range(_NT):
        xs = _S + k * _TI
        xc = (cot_ref[0, pl.ds(xs, _TI), :]
              + sxt_ref[0, 0, pl.ds(xs, _TI), :])
        xb = xc.astype(jnp.bfloat16)
        g = lax.dot_general(xb, yb, (((1,), (0,)), ((), ())),
                            preferred_element_type=jnp.float32)
        t = n2[None, :] - 2.0 * g
        m = jnp.min(t, axis=1)
        out_ref[0, 0, _S // _TI + k, :] = m + jnp.sum(xc * xc, axis=1)


def kernel(registration_pred, registration_gt, coords, wandb):
    pc1 = coords + registration_gt
    pc2 = coords + registration_pred
    cl = jnp.stack([pc1, pc2])
    clb = cl.astype(jnp.bfloat16).astype(jnp.float32)
    nr = jnp.sum(cl * cl, axis=2)

    sel = jnp.stack([registration_gt, registration_pred])
    sel_t = jnp.transpose(sel, (0, 1, 3, 2))
    coords_t = jnp.transpose(coords, (0, 2, 1))
    tc = pl.pallas_call(
        _tc_body,
        grid=(2, _B),
        in_specs=[
            pl.BlockSpec((1, _C, _N), lambda d, b: (b, 0, 0)),
            pl.BlockSpec((1, _N, _C), lambda d, b: (b, 0, 0)),
            pl.BlockSpec((1, 1, _N, _C), lambda d, b: (d, b, 0, 0)),
            pl.BlockSpec((1, 1, _C, _N), lambda d, b: (1 - d, b, 0, 0)),
        ],
        out_specs=pl.BlockSpec((1, 1, _N // _TI, _TI),
                               lambda d, b: (d, b, 0, 0)),
        out_shape=jax.ShapeDtypeStruct((2, _B, _N // _TI, _TI), jnp.float32),
    )(coords, coords_t, sel_t, sel)

    sc_d1, sc_d2 = _chamfer_sc(clb, nr)

    tc = tc.reshape(2, _B, _N)
    d1 = jnp.concatenate([sc_d1, tc[0, :, _S:]], axis=1)
    d2 = jnp.concatenate([sc_d2, tc[1, :, _S:]], axis=1)
    return jnp.mean(jnp.sum(d1, axis=1)) + jnp.mean(jnp.sum(d2, axis=1))


# speedup vs baseline: 1.0028x; 1.0028x over previous
import jax
import jax.numpy as jnp
from jax import lax
from jax.experimental import pallas as pl
from jax.experimental.pallas import tpu as pltpu
from jax.experimental.pallas import tpu_sc as plsc

_B, _C, _N = 2, 3, 4096
_S = 512
_TI = 512
_NT = (_N - _S) // _TI
_NC, _NS, _L = 2, 16, 16
_NW = _NC * _NS
_SPW = _S // _NW
_NJ = _N // _L
_HB = 8


def _sc_body(cl_hbm, nr_hbm, d1_hbm, d2_hbm,
             y_v, yn_v, x_v, xn_v, o_v, m_v):
    wid = lax.axis_index("s") * _NC + lax.axis_index("c")
    x0 = wid * _SPW
    lane = lax.iota(jnp.int32, _L)

    for d in range(2):
        for b in range(_B):
            pltpu.sync_copy(cl_hbm.at[1 - d, b], y_v)
            pltpu.sync_copy(nr_hbm.at[1 - d, b], yn_v)
            for c in range(_C):
                pltpu.sync_copy(cl_hbm.at[d, b, c, pl.ds(x0, _SPW)],
                                x_v.at[c])
            pltpu.sync_copy(nr_hbm.at[d, b, pl.ds(x0, _SPW)], xn_v)

            vx0 = x_v[0, :]
            vx1 = x_v[1, :]
            vx2 = x_v[2, :]
            for half in range(_SPW // _HB):
                xm = [tuple(jnp.broadcast_to(vx[half * _HB + ii] * (-2.0),
                                             (_L,))
                            for vx in (vx0, vx1, vx2))
                      for ii in range(_HB)]

                def jbody(off, mins):
                    y0 = y_v[0, pl.ds(off, _L)]
                    y1 = y_v[1, pl.ds(off, _L)]
                    y2 = y_v[2, pl.ds(off, _L)]
                    yn = yn_v[pl.ds(off, _L)]
                    return tuple(
                        jnp.minimum(mins[ii],
                                    yn + xm[ii][0] * y0
                                    + xm[ii][1] * y1
                                    + xm[ii][2] * y2)
                        for ii in range(_HB))

                init = tuple(jnp.full((_L,), 3.0e38, jnp.float32)
                             for _ in range(_HB))
                mins = plsc.parallel_loop(
                    0, _N, step=_L, unroll=2, carry=init)(jbody)
                for ii in range(_HB):
                    m_v[pl.ds((half * _HB + ii) * _L, _L)] = mins[ii]
            red = None
            for l in range(_L):
                col = plsc.load_gather(m_v, [lane * _L + l])
                red = col if red is None else jnp.minimum(red, col)
            o_v[...] = red + xn_v[...]
            out = d1_hbm if d == 0 else d2_hbm
            pltpu.sync_copy(o_v, out.at[b, pl.ds(x0, _SPW)])


_chamfer_sc = pl.kernel(
    _sc_body,
    out_type=(jax.ShapeDtypeStruct((_B, _S), jnp.float32),
              jax.ShapeDtypeStruct((_B, _S), jnp.float32)),
    mesh=plsc.VectorSubcoreMesh(core_axis_name="c", subcore_axis_name="s"),
    compiler_params=pltpu.CompilerParams(needs_layout_passes=False,
                                         skip_device_barrier=True),
    cost_estimate=pl.CostEstimate(flops=8 * _S * _N * _B,
                                  transcendentals=0,
                                  bytes_accessed=2 * _B * 4 * _C * _N * 4),
    scratch_types=[
        pltpu.VMEM((_C, _N), jnp.float32),
        pltpu.VMEM((_N,), jnp.float32),
        pltpu.VMEM((_C, _SPW), jnp.float32),
        pltpu.VMEM((_SPW,), jnp.float32),
        pltpu.VMEM((_SPW,), jnp.float32),
        pltpu.VMEM((_L * _L,), jnp.float32),
    ],
)


def _tc_body(co_ref, cot_ref, sxt_ref, sy_ref, out_ref):
    yc = co_ref[0] + sy_ref[0, 0]
    yb = yc.astype(jnp.bfloat16)
    n2 = jnp.sum(yc * yc, axis=0)
    for k in range(_NT):
        xs = _S + k * _TI
        xc = (cot_ref[0, pl.ds(xs, _TI), :]
              + sxt_ref[0, 0, pl.ds(xs, _TI), :])
        xb = xc.astype(jnp.bfloat16)
        g = lax.dot_general(xb, yb, (((1,), (0,)), ((), ())),
                            preferred_element_type=jnp.float32)
        t = n2[None, :] - 2.0 * g
        m = jnp.min(t, axis=1)
        out_ref[0, 0, _S // _TI + k, :] = m + jnp.sum(xc * xc, axis=1)


def kernel(registration_pred, registration_gt, coords, wandb):
    pc1 = coords + registration_gt
    pc2 = coords + registration_pred
    cl = jnp.stack([pc1, pc2])
    clb = cl.astype(jnp.bfloat16).astype(jnp.float32)
    nr = jnp.sum(cl * cl, axis=2)

    sel = jnp.stack([registration_gt, registration_pred])
    sel_t = jnp.transpose(sel, (0, 1, 3, 2))
    coords_t = jnp.transpose(coords, (0, 2, 1))
    tc = pl.pallas_call(
        _tc_body,
        grid=(2, _B),
        in_specs=[
            pl.BlockSpec((1, _C, _N), lambda d, b: (b, 0, 0)),
            pl.BlockSpec((1, _N, _C), lambda d, b: (b, 0, 0)),
            pl.BlockSpec((1, 1, _N, _C), lambda d, b: (d, b, 0, 0)),
            pl.BlockSpec((1, 1, _C, _N), lambda d, b: (1 - d, b, 0, 0)),
        ],
        out_specs=pl.BlockSpec((1, 1, _N // _TI, _TI),
                               lambda d, b: (d, b, 0, 0)),
        out_shape=jax.ShapeDtypeStruct((2, _B, _N // _TI, _TI), jnp.float32),
        cost_estimate=pl.CostEstimate(flops=8 * (_N - _S) * _N * _B,
                                      transcendentals=0,
                                      bytes_accessed=4 * _B * _C * _N * 4),
    )(coords, coords_t, sel_t, sel)

    sc_d1, sc_d2 = _chamfer_sc(clb, nr)

    tc = tc.reshape(2, _B, _N)
    d1 = jnp.concatenate([sc_d1, tc[0, :, _S:]], axis=1)
    d2 = jnp.concatenate([sc_d2, tc[1, :, _S:]], axis=1)
    return jnp.mean(jnp.sum(d1, axis=1)) + jnp.mean(jnp.sum(d2, axis=1))


# speedup vs baseline: 1.0166x; 1.0138x over previous
import jax
import jax.numpy as jnp
from jax import lax
from jax.experimental import pallas as pl
from jax.experimental.pallas import tpu as pltpu
from jax.experimental.pallas import tpu_sc as plsc

_B, _C, _N = 2, 3, 4096
_S = 512
_TI = 512
_NT = (_N - _S) // _TI
_NC, _NS, _L = 2, 16, 16
_NW = _NC * _NS
_SPW = _S // _NW
_NJ = _N // _L
_HB = 8


def _sc_body(cl_hbm, nr_hbm, d1_hbm, d2_hbm, cl_v, nr_v, o_v, m_v):
    wid = lax.axis_index("s") * _NC + lax.axis_index("c")
    x0 = wid * _SPW
    lane = lax.iota(jnp.int32, _L)

    pltpu.sync_copy(cl_hbm, cl_v)
    pltpu.sync_copy(nr_hbm, nr_v)

    for d in range(2):
        for b in range(_B):
            vx0 = cl_v[d, b, 0, pl.ds(x0, _SPW)]
            vx1 = cl_v[d, b, 1, pl.ds(x0, _SPW)]
            vx2 = cl_v[d, b, 2, pl.ds(x0, _SPW)]
            for half in range(_SPW // _HB):
                xm = [tuple(jnp.broadcast_to(vx[half * _HB + ii] * (-2.0),
                                             (_L,))
                            for vx in (vx0, vx1, vx2))
                      for ii in range(_HB)]

                def jbody(off, mins):
                    y0 = cl_v[1 - d, b, 0, pl.ds(off, _L)]
                    y1 = cl_v[1 - d, b, 1, pl.ds(off, _L)]
                    y2 = cl_v[1 - d, b, 2, pl.ds(off, _L)]
                    yn = nr_v[1 - d, b, pl.ds(off, _L)]
                    return tuple(
                        jnp.minimum(mins[ii],
                                    yn + xm[ii][0] * y0
                                    + xm[ii][1] * y1
                                    + xm[ii][2] * y2)
                        for ii in range(_HB))

                init = tuple(jnp.full((_L,), 3.0e38, jnp.float32)
                             for _ in range(_HB))
                mins = plsc.parallel_loop(
                    0, _N, step=_L, unroll=2, carry=init)(jbody)
                for ii in range(_HB):
                    m_v[pl.ds((half * _HB + ii) * _L, _L)] = mins[ii]
            red = None
            for l in range(_L):
                col = plsc.load_gather(m_v, [lane * _L + l])
                red = col if red is None else jnp.minimum(red, col)
            o_v[...] = red + nr_v[d, b, pl.ds(x0, _SPW)]
            out = d1_hbm if d == 0 else d2_hbm
            pltpu.sync_copy(o_v, out.at[b, pl.ds(x0, _SPW)])


_chamfer_sc = pl.kernel(
    _sc_body,
    out_type=(jax.ShapeDtypeStruct((_B, _S), jnp.float32),
              jax.ShapeDtypeStruct((_B, _S), jnp.float32)),
    mesh=plsc.VectorSubcoreMesh(core_axis_name="c", subcore_axis_name="s"),
    compiler_params=pltpu.CompilerParams(needs_layout_passes=False,
                                         skip_device_barrier=True),
    cost_estimate=pl.CostEstimate(flops=8 * _S * _N * _B,
                                  transcendentals=0,
                                  bytes_accessed=2 * _B * 4 * _C * _N * 4),
    scratch_types=[
        pltpu.VMEM((2, _B, _C, _N), jnp.float32),
        pltpu.VMEM((2, _B, _N), jnp.float32),
        pltpu.VMEM((_SPW,), jnp.float32),
        pltpu.VMEM((_L * _L,), jnp.float32),
    ],
)


def _tc_body(co_ref, cot_ref, sxt_ref, sy_ref, out_ref):
    yc = co_ref[0] + sy_ref[0, 0]
    yb = yc.astype(jnp.bfloat16)
    n2 = jnp.sum(yc * yc, axis=0)
    for k in range(_NT):
        xs = _S + k * _TI
        xc = (cot_ref[0, pl.ds(xs, _TI), :]
              + sxt_ref[0, 0, pl.ds(xs, _TI), :])
        xb = xc.astype(jnp.bfloat16)
        g = lax.dot_general(xb, yb, (((1,), (0,)), ((), ())),
                            preferred_element_type=jnp.float32)
        t = n2[None, :] - 2.0 * g
        m = jnp.min(t, axis=1)
        out_ref[0, 0, _S // _TI + k, :] = m + jnp.sum(xc * xc, axis=1)


def kernel(registration_pred, registration_gt, coords, wandb):
    pc1 = coords + registration_gt
    pc2 = coords + registration_pred
    cl = jnp.stack([pc1, pc2])
    clb = cl.astype(jnp.bfloat16).astype(jnp.float32)
    nr = jnp.sum(cl * cl, axis=2)

    sel = jnp.stack([registration_gt, registration_pred])
    sel_t = jnp.transpose(sel, (0, 1, 3, 2))
    coords_t = jnp.transpose(coords, (0, 2, 1))
    tc = pl.pallas_call(
        _tc_body,
        grid=(2, _B),
        in_specs=[
            pl.BlockSpec((1, _C, _N), lambda d, b: (b, 0, 0)),
            pl.BlockSpec((1, _N, _C), lambda d, b: (b, 0, 0)),
            pl.BlockSpec((1, 1, _N, _C), lambda d, b: (d, b, 0, 0)),
            pl.BlockSpec((1, 1, _C, _N), lambda d, b: (1 - d, b, 0, 0)),
        ],
        out_specs=pl.BlockSpec((1, 1, _N // _TI, _TI),
                               lambda d, b: (d, b, 0, 0)),
        out_shape=jax.ShapeDtypeStruct((2, _B, _N // _TI, _TI), jnp.float32),
        cost_estimate=pl.CostEstimate(flops=8 * (_N - _S) * _N * _B,
                                      transcendentals=0,
                                      bytes_accessed=4 * _B * _C * _N * 4),
    )(coords, coords_t, sel_t, sel)

    sc_d1, sc_d2 = _chamfer_sc(clb, nr)

    tc = tc.reshape(2, _B, _N)
    d1 = jnp.concatenate([sc_d1, tc[0, :, _S:]], axis=1)
    d2 = jnp.concatenate([sc_d2, tc[1, :, _S:]], axis=1)
    return jnp.mean(jnp.sum(d1, axis=1)) + jnp.mean(jnp.sum(d2, axis=1))
